# Initial kernel scaffold; baseline (speedup 1.0000x reference)
import jax
import jax.numpy as jnp
from jax.experimental import pallas as pl


def kernel(x, edge_index, target_index, W0, b0, gamma0, beta0, W1, b1, Wf1, bf1, Wf2, bf2):
    raise NotImplementedError("write your pallas kernel here")


# speedup vs baseline: 19.0130x; 19.0130x over previous
import functools

import jax
import jax.numpy as jnp
from jax import lax
from jax.experimental import pallas as pl
from jax.experimental.pallas import tpu as pltpu
from jax.experimental.pallas import tpu_sc as plsc

N = 10000
NP = 10240
E = 320000
D = 128
B = 4096

NC = 2
NS = 16
NW = NC * NS
EPT = E // NW
CH = 125
NCH = EPT // CH
SB = 16
NSB = NCH // SB
RPT = NP // NS
ZB = 128
BPT = B // NW

_MESH = plsc.VectorSubcoreMesh(core_axis_name="c", subcore_axis_name="s",
                               num_cores=NC, num_subcores=NS)


def _deg_body(dst_hbm, ones_hbm, zeros_hbm, out_hbm, degacc, dstv, ones_buf):
    cid = lax.axis_index("c")
    sid = lax.axis_index("s")
    wid = sid * NC + cid
    pltpu.sync_copy(dst_hbm.at[wid], dstv)
    pltpu.sync_copy(ones_hbm, ones_buf)
    pltpu.sync_copy(zeros_hbm.at[pl.ds(sid * RPT, RPT)],
                    degacc.at[pl.ds(sid * RPT, RPT)])
    plsc.subcore_barrier()

    def add_body(j, _):
        pltpu.sync_copy(ones_buf, degacc.at[dstv.at[j]], add=True)
        return 0

    lax.fori_loop(0, NCH, add_body, 0)
    plsc.subcore_barrier()
    pltpu.sync_copy(degacc.at[pl.ds(sid * RPT, RPT)],
                    out_hbm.at[cid].at[pl.ds(sid * RPT, RPT)])


def _agg_body(g_hbm, src_hbm, dst_hbm, zeros_hbm, out_hbm,
              acc, srcv, dstv, rows0, sem0):
    cid = lax.axis_index("c")
    sid = lax.axis_index("s")
    wid = sid * NC + cid
    pltpu.sync_copy(src_hbm.at[wid], srcv)
    pltpu.sync_copy(dst_hbm.at[wid], dstv)

    pltpu.sync_copy(zeros_hbm.at[pl.ds(sid * RPT, RPT)],
                    acc.at[pl.ds(sid * RPT, RPT)])
    plsc.subcore_barrier()

    def chunk_body(j, _):
        pltpu.async_copy(g_hbm.at[srcv.at[j]], rows0, sem0).wait()
        pltpu.sync_copy(rows0, acc.at[dstv.at[j]], add=True)
        return 0

    lax.fori_loop(0, NCH, chunk_body, 0)
    plsc.subcore_barrier()
    pltpu.sync_copy(acc.at[pl.ds(sid * RPT, RPT)],
                    out_hbm.at[cid].at[pl.ds(sid * RPT, RPT)])


def _tgt_gather_body(tab_hbm, idx_hbm, out_hbm, idxv, rows, sem):
    cid = lax.axis_index("c")
    sid = lax.axis_index("s")
    wid = sid * NC + cid
    base = wid * BPT
    pltpu.sync_copy(idx_hbm.at[pl.ds(base, BPT)], idxv)
    pltpu.async_copy(tab_hbm.at[idxv], rows, sem).wait()
    pltpu.sync_copy(rows, out_hbm.at[pl.ds(base, BPT)])


_DEG_SCRATCH = [
    pltpu.VMEM_SHARED((NP, D), jnp.float32),
    pltpu.VMEM((NCH, CH), jnp.int32),
    pltpu.VMEM((CH, D), jnp.float32),
]
_AGG_SCRATCH = [
    pltpu.VMEM_SHARED((NP, D), jnp.float32),
    pltpu.VMEM((NCH, CH), jnp.int32),
    pltpu.VMEM((NCH, CH), jnp.int32),
    pltpu.VMEM((CH, D), jnp.float32),
    pltpu.SemaphoreType.DMA,
]
_TGT_SCRATCH = [
    pltpu.VMEM((BPT,), jnp.int32),
    pltpu.VMEM((BPT, D), jnp.float32),
    pltpu.SemaphoreType.DMA,
]

_deg_kernel = pl.kernel(
    _deg_body, out_type=jax.ShapeDtypeStruct((NC, NP, D), jnp.float32),
    mesh=_MESH, scratch_types=_DEG_SCRATCH)
_agg_kernel = pl.kernel(
    _agg_body, out_type=jax.ShapeDtypeStruct((NC, NP, D), jnp.float32),
    mesh=_MESH, scratch_types=_AGG_SCRATCH)
_tgt_gather_kernel = pl.kernel(
    _tgt_gather_body, out_type=jax.ShapeDtypeStruct((B, D), jnp.float32),
    mesh=_MESH, scratch_types=_TGT_SCRATCH)


RB = 1024
GN = NP // RB
FB = 1024
GF = B // FB


def _dinv_body(degp_ref, dinvb_ref):
    dinvb_ref[...] = lax.rsqrt(degp_ref[0] + degp_ref[1] + 1.0)


_dinv_call = pl.pallas_call(
    _dinv_body,
    grid=(GN,),
    in_specs=[pl.BlockSpec((NC, RB, D), lambda i: (0, i, 0))],
    out_specs=pl.BlockSpec((RB, D), lambda i: (i, 0)),
    out_shape=jax.ShapeDtypeStruct((NP, D), jnp.float32),
)


def _mm_scale_body(x_ref, w_ref, dinvb_ref, out_ref):
    h = jnp.dot(x_ref[...], w_ref[...], preferred_element_type=jnp.float32)
    out_ref[...] = h * dinvb_ref[...]


_mm_scale_call = pl.pallas_call(
    _mm_scale_body,
    grid=(GN,),
    in_specs=[
        pl.BlockSpec((RB, D), lambda i: (i, 0)),
        pl.BlockSpec((D, D), lambda i: (0, 0)),
        pl.BlockSpec((RB, D), lambda i: (i, 0)),
    ],
    out_specs=pl.BlockSpec((RB, D), lambda i: (i, 0)),
    out_shape=jax.ShapeDtypeStruct((NP, D), jnp.float32),
)


def _stats_body(p_ref, g_ref, dinvb_ref, b_ref, out_ref, sum_ref, sq_ref):
    @pl.when(pl.program_id(0) == 0)
    def _():
        sum_ref[...] = jnp.zeros_like(sum_ref)
        sq_ref[...] = jnp.zeros_like(sq_ref)

    val = (p_ref[0] + p_ref[1] + g_ref[...]) * dinvb_ref[...] + b_ref[...]
    out_ref[...] = val
    row = (pl.program_id(0) * RB
           + lax.broadcasted_iota(jnp.int32, (RB, D), 0))
    vm = jnp.where(row < N, val, 0.0)
    sum_ref[...] += jnp.sum(vm, axis=0, keepdims=True)
    sq_ref[...] += jnp.sum(vm * vm, axis=0, keepdims=True)


_stats_call = pl.pallas_call(
    _stats_body,
    grid=(GN,),
    in_specs=[
        pl.BlockSpec((NC, RB, D), lambda i: (0, i, 0)),
        pl.BlockSpec((RB, D), lambda i: (i, 0)),
        pl.BlockSpec((RB, D), lambda i: (i, 0)),
        pl.BlockSpec((1, D), lambda i: (0, 0)),
    ],
    out_specs=[
        pl.BlockSpec((RB, D), lambda i: (i, 0)),
        pl.BlockSpec((1, D), lambda i: (0, 0)),
        pl.BlockSpec((1, D), lambda i: (0, 0)),
    ],
    out_shape=[
        jax.ShapeDtypeStruct((NP, D), jnp.float32),
        jax.ShapeDtypeStruct((1, D), jnp.float32),
        jax.ShapeDtypeStruct((1, D), jnp.float32),
    ],
)


def _bn_mm_body(x_ref, sum_ref, sq_ref, gamma_ref, beta_ref, w_ref,
                dinvb_ref, out_ref):
    mean = sum_ref[...] / N
    var = sq_ref[...] / N - mean * mean
    scale = gamma_ref[...] * lax.rsqrt(var + 1e-5)
    shift = beta_ref[...] - mean * scale
    h = jnp.maximum(x_ref[...] * scale + shift, 0.0)
    out_ref[...] = jnp.dot(h, w_ref[...],
                           preferred_element_type=jnp.float32) * dinvb_ref[...]


_bn_mm_call = pl.pallas_call(
    _bn_mm_body,
    grid=(GN,),
    in_specs=[
        pl.BlockSpec((RB, D), lambda i: (i, 0)),
        pl.BlockSpec((1, D), lambda i: (0, 0)),
        pl.BlockSpec((1, D), lambda i: (0, 0)),
        pl.BlockSpec((1, D), lambda i: (0, 0)),
        pl.BlockSpec((1, D), lambda i: (0, 0)),
        pl.BlockSpec((D, D), lambda i: (0, 0)),
        pl.BlockSpec((RB, D), lambda i: (i, 0)),
    ],
    out_specs=pl.BlockSpec((RB, D), lambda i: (i, 0)),
    out_shape=jax.ShapeDtypeStruct((NP, D), jnp.float32),
)


def _out1_body(p_ref, g_ref, dinvb_ref, b_ref, out_ref):
    out_ref[...] = ((p_ref[0] + p_ref[1] + g_ref[...]) * dinvb_ref[...]
                    + b_ref[...])


_out1_call = pl.pallas_call(
    _out1_body,
    grid=(GN,),
    in_specs=[
        pl.BlockSpec((NC, RB, D), lambda i: (0, i, 0)),
        pl.BlockSpec((RB, D), lambda i: (i, 0)),
        pl.BlockSpec((RB, D), lambda i: (i, 0)),
        pl.BlockSpec((1, D), lambda i: (0, 0)),
    ],
    out_specs=pl.BlockSpec((RB, D), lambda i: (i, 0)),
    out_shape=jax.ShapeDtypeStruct((NP, D), jnp.float32),
)


def _ffn_body(x_ref, w1_ref, b1_ref, w2_ref, b2_ref, out_ref):
    f = jnp.maximum(
        jnp.dot(x_ref[...], w1_ref[...], preferred_element_type=jnp.float32)
        + b1_ref[...], 0.0)
    out_ref[...] = (
        jnp.dot(f, w2_ref[...], preferred_element_type=jnp.float32)
        + b2_ref[...])


_ffn_call = pl.pallas_call(
    _ffn_body,
    grid=(GF,),
    in_specs=[
        pl.BlockSpec((FB, D), lambda i: (i, 0)),
        pl.BlockSpec((D, D), lambda i: (0, 0)),
        pl.BlockSpec((1, D), lambda i: (0, 0)),
        pl.BlockSpec((D, D), lambda i: (0, 0)),
        pl.BlockSpec((1, D), lambda i: (0, 0)),
    ],
    out_specs=pl.BlockSpec((FB, D), lambda i: (i, 0)),
    out_shape=jax.ShapeDtypeStruct((B, D), jnp.float32),
)


def kernel(x, edge_index, target_index, W0, b0, gamma0, beta0, W1, b1,
           Wf1, bf1, Wf2, bf2):
    ei = edge_index.astype(jnp.int32)
    src3 = ei[0].reshape(NW, NCH, CH)
    dst3 = ei[1].reshape(NW, NCH, CH)
    tgt = target_index.astype(jnp.int32)
    xp = jnp.pad(x, ((0, NP - N), (0, 0)))
    b0r = b0.reshape(1, D)
    b1r = b1.reshape(1, D)
    gammar = gamma0.reshape(1, D)
    betar = beta0.reshape(1, D)
    bf1r = bf1.reshape(1, D)
    bf2r = bf2.reshape(1, D)

    zeros = jnp.zeros((NP, D), jnp.float32)

    ones_rows = jnp.ones((CH, D), jnp.float32)
    degp = _deg_kernel(dst3, ones_rows, zeros)
    dinvb = _dinv_call(degp)
    g0 = _mm_scale_call(xp, W0, dinvb)
    p0 = _agg_kernel(g0, src3, dst3, zeros)
    out0, s_sum, s_sq = _stats_call(p0, g0, dinvb, b0r)
    g1 = _bn_mm_call(out0, s_sum, s_sq, gammar, betar, W1, dinvb)
    p1 = _agg_kernel(g1, src3, dst3, zeros)
    out1 = _out1_call(p1, g1, dinvb, b1r)
    pooled = _tgt_gather_kernel(out1, tgt)
    return _ffn_call(pooled, Wf1, bf1r, Wf2, bf2r)


# speedup vs baseline: 25.4384x; 1.3379x over previous
import functools

import jax
import jax.numpy as jnp
from jax import lax
from jax.experimental import pallas as pl
from jax.experimental.pallas import tpu as pltpu
from jax.experimental.pallas import tpu_sc as plsc

N = 10000
NP = 10240
E = 320000
D = 128
B = 4096

NC = 2
NS = 16
NW = NC * NS
EPT = E // NW
CH = 125
NCH = EPT // CH
NH = 2
HC = NCH // NH
RPT = NP // NS
ZB = 128
BPT = B // NW

_MESH = plsc.VectorSubcoreMesh(core_axis_name="c", subcore_axis_name="s",
                               num_cores=NC, num_subcores=NS)


def _deg_body(dst_hbm, ones_hbm, zeros_hbm, out_hbm, degacc, dstv, ones_buf):
    cid = lax.axis_index("c")
    sid = lax.axis_index("s")
    wid = sid * NC + cid
    pltpu.sync_copy(dst_hbm.at[wid], dstv)
    pltpu.sync_copy(ones_hbm, ones_buf)
    pltpu.sync_copy(zeros_hbm.at[pl.ds(sid * RPT, RPT)],
                    degacc.at[pl.ds(sid * RPT, RPT)])
    plsc.subcore_barrier()

    def add_body(j, _):
        pltpu.sync_copy(ones_buf, degacc.at[dstv.at[j]], add=True)
        return 0

    lax.fori_loop(0, NCH, add_body, 0)
    plsc.subcore_barrier()
    pltpu.sync_copy(degacc.at[pl.ds(sid * RPT, RPT)],
                    out_hbm.at[cid].at[pl.ds(sid * RPT, RPT)])


def _agg_body(g_hbm, src_hbm, dst_hbm, zeros_hbm, out_hbm,
              acc, srcv, dstv, rows0, rows1, sem0, sem1):
    cid = lax.axis_index("c")
    sid = lax.axis_index("s")
    wid = sid * NC + cid
    pltpu.sync_copy(zeros_hbm.at[pl.ds(sid * RPT, RPT)],
                    acc.at[pl.ds(sid * RPT, RPT)])
    plsc.subcore_barrier()

    for h in range(NH):
        pltpu.sync_copy(src_hbm.at[wid].at[h], srcv)
        pltpu.sync_copy(dst_hbm.at[wid].at[h], dstv)
        pltpu.async_copy(g_hbm.at[srcv.at[0]], rows0, sem0)

        def pair_body(k, _):
            pltpu.async_copy(g_hbm.at[srcv.at[2 * k + 1]], rows1, sem1)
            pltpu.make_async_copy(g_hbm.at[srcv.at[2 * k]], rows0, sem0).wait()
            pltpu.sync_copy(rows0, acc.at[dstv.at[2 * k]], add=True)

            @pl.when(k < HC // 2 - 1)
            def _():
                pltpu.async_copy(g_hbm.at[srcv.at[2 * k + 2]], rows0, sem0)

            pltpu.make_async_copy(g_hbm.at[srcv.at[2 * k + 1]], rows1,
                                  sem1).wait()
            pltpu.sync_copy(rows1, acc.at[dstv.at[2 * k + 1]], add=True)
            return 0

        lax.fori_loop(0, HC // 2, pair_body, 0)
    plsc.subcore_barrier()
    pltpu.sync_copy(acc.at[pl.ds(sid * RPT, RPT)],
                    out_hbm.at[cid].at[pl.ds(sid * RPT, RPT)])


def _tgt_gather_body(tab_hbm, idx_hbm, out_hbm, idxv, rows, sem):
    cid = lax.axis_index("c")
    sid = lax.axis_index("s")
    wid = sid * NC + cid
    base = wid * BPT
    pltpu.sync_copy(idx_hbm.at[pl.ds(base, BPT)], idxv)
    pltpu.async_copy(tab_hbm.at[idxv], rows, sem).wait()
    pltpu.sync_copy(rows, out_hbm.at[pl.ds(base, BPT)])


_DEG_SCRATCH = [
    pltpu.VMEM_SHARED((NP, D), jnp.float32),
    pltpu.VMEM((NCH, CH), jnp.int32),
    pltpu.VMEM((CH, D), jnp.float32),
]
_AGG_SCRATCH = [
    pltpu.VMEM_SHARED((NP, D), jnp.float32),
    pltpu.VMEM((HC, CH), jnp.int32),
    pltpu.VMEM((HC, CH), jnp.int32),
    pltpu.VMEM((CH, D), jnp.float32),
    pltpu.VMEM((CH, D), jnp.float32),
    pltpu.SemaphoreType.DMA,
    pltpu.SemaphoreType.DMA,
]
_TGT_SCRATCH = [
    pltpu.VMEM((BPT,), jnp.int32),
    pltpu.VMEM((BPT, D), jnp.float32),
    pltpu.SemaphoreType.DMA,
]

_deg_kernel = pl.kernel(
    _deg_body, out_type=jax.ShapeDtypeStruct((NC, NP, D), jnp.float32),
    mesh=_MESH, scratch_types=_DEG_SCRATCH)
_agg_kernel = pl.kernel(
    _agg_body, out_type=jax.ShapeDtypeStruct((NC, NP, D), jnp.float32),
    mesh=_MESH, scratch_types=_AGG_SCRATCH)
_tgt_gather_kernel = pl.kernel(
    _tgt_gather_body, out_type=jax.ShapeDtypeStruct((B, D), jnp.float32),
    mesh=_MESH, scratch_types=_TGT_SCRATCH)


RB = 1024
GN = NP // RB
FB = 1024
GF = B // FB


def _dinv_body(degp_ref, dinvb_ref):
    dinvb_ref[...] = lax.rsqrt(degp_ref[0] + degp_ref[1] + 1.0)


_dinv_call = pl.pallas_call(
    _dinv_body,
    grid=(GN,),
    in_specs=[pl.BlockSpec((NC, RB, D), lambda i: (0, i, 0))],
    out_specs=pl.BlockSpec((RB, D), lambda i: (i, 0)),
    out_shape=jax.ShapeDtypeStruct((NP, D), jnp.float32),
)


def _mm_scale_body(x_ref, w_ref, dinvb_ref, out_ref):
    h = jnp.dot(x_ref[...], w_ref[...], preferred_element_type=jnp.float32)
    out_ref[...] = h * dinvb_ref[...]


_mm_scale_call = pl.pallas_call(
    _mm_scale_body,
    grid=(GN,),
    in_specs=[
        pl.BlockSpec((RB, D), lambda i: (i, 0)),
        pl.BlockSpec((D, D), lambda i: (0, 0)),
        pl.BlockSpec((RB, D), lambda i: (i, 0)),
    ],
    out_specs=pl.BlockSpec((RB, D), lambda i: (i, 0)),
    out_shape=jax.ShapeDtypeStruct((NP, D), jnp.float32),
)


def _stats_body(p_ref, g_ref, dinvb_ref, b_ref, out_ref, sum_ref, sq_ref):
    @pl.when(pl.program_id(0) == 0)
    def _():
        sum_ref[...] = jnp.zeros_like(sum_ref)
        sq_ref[...] = jnp.zeros_like(sq_ref)

    val = (p_ref[0] + p_ref[1] + g_ref[...]) * dinvb_ref[...] + b_ref[...]
    out_ref[...] = val
    row = (pl.program_id(0) * RB
           + lax.broadcasted_iota(jnp.int32, (RB, D), 0))
    vm = jnp.where(row < N, val, 0.0)
    sum_ref[...] += jnp.sum(vm, axis=0, keepdims=True)
    sq_ref[...] += jnp.sum(vm * vm, axis=0, keepdims=True)


_stats_call = pl.pallas_call(
    _stats_body,
    grid=(GN,),
    in_specs=[
        pl.BlockSpec((NC, RB, D), lambda i: (0, i, 0)),
        pl.BlockSpec((RB, D), lambda i: (i, 0)),
        pl.BlockSpec((RB, D), lambda i: (i, 0)),
        pl.BlockSpec((1, D), lambda i: (0, 0)),
    ],
    out_specs=[
        pl.BlockSpec((RB, D), lambda i: (i, 0)),
        pl.BlockSpec((1, D), lambda i: (0, 0)),
        pl.BlockSpec((1, D), lambda i: (0, 0)),
    ],
    out_shape=[
        jax.ShapeDtypeStruct((NP, D), jnp.float32),
        jax.ShapeDtypeStruct((1, D), jnp.float32),
        jax.ShapeDtypeStruct((1, D), jnp.float32),
    ],
)


def _bn_mm_body(x_ref, sum_ref, sq_ref, gamma_ref, beta_ref, w_ref,
                dinvb_ref, out_ref):
    mean = sum_ref[...] / N
    var = sq_ref[...] / N - mean * mean
    scale = gamma_ref[...] * lax.rsqrt(var + 1e-5)
    shift = beta_ref[...] - mean * scale
    h = jnp.maximum(x_ref[...] * scale + shift, 0.0)
    out_ref[...] = jnp.dot(h, w_ref[...],
                           preferred_element_type=jnp.float32) * dinvb_ref[...]


_bn_mm_call = pl.pallas_call(
    _bn_mm_body,
    grid=(GN,),
    in_specs=[
        pl.BlockSpec((RB, D), lambda i: (i, 0)),
        pl.BlockSpec((1, D), lambda i: (0, 0)),
        pl.BlockSpec((1, D), lambda i: (0, 0)),
        pl.BlockSpec((1, D), lambda i: (0, 0)),
        pl.BlockSpec((1, D), lambda i: (0, 0)),
        pl.BlockSpec((D, D), lambda i: (0, 0)),
        pl.BlockSpec((RB, D), lambda i: (i, 0)),
    ],
    out_specs=pl.BlockSpec((RB, D), lambda i: (i, 0)),
    out_shape=jax.ShapeDtypeStruct((NP, D), jnp.float32),
)


def _out1_body(p_ref, g_ref, dinvb_ref, b_ref, out_ref):
    out_ref[...] = ((p_ref[0] + p_ref[1] + g_ref[...]) * dinvb_ref[...]
                    + b_ref[...])


_out1_call = pl.pallas_call(
    _out1_body,
    grid=(GN,),
    in_specs=[
        pl.BlockSpec((NC, RB, D), lambda i: (0, i, 0)),
        pl.BlockSpec((RB, D), lambda i: (i, 0)),
        pl.BlockSpec((RB, D), lambda i: (i, 0)),
        pl.BlockSpec((1, D), lambda i: (0, 0)),
    ],
    out_specs=pl.BlockSpec((RB, D), lambda i: (i, 0)),
    out_shape=jax.ShapeDtypeStruct((NP, D), jnp.float32),
)


def _ffn_body(x_ref, w1_ref, b1_ref, w2_ref, b2_ref, out_ref):
    f = jnp.maximum(
        jnp.dot(x_ref[...], w1_ref[...], preferred_element_type=jnp.float32)
        + b1_ref[...], 0.0)
    out_ref[...] = (
        jnp.dot(f, w2_ref[...], preferred_element_type=jnp.float32)
        + b2_ref[...])


_ffn_call = pl.pallas_call(
    _ffn_body,
    grid=(GF,),
    in_specs=[
        pl.BlockSpec((FB, D), lambda i: (i, 0)),
        pl.BlockSpec((D, D), lambda i: (0, 0)),
        pl.BlockSpec((1, D), lambda i: (0, 0)),
        pl.BlockSpec((D, D), lambda i: (0, 0)),
        pl.BlockSpec((1, D), lambda i: (0, 0)),
    ],
    out_specs=pl.BlockSpec((FB, D), lambda i: (i, 0)),
    out_shape=jax.ShapeDtypeStruct((B, D), jnp.float32),
)


def kernel(x, edge_index, target_index, W0, b0, gamma0, beta0, W1, b1,
           Wf1, bf1, Wf2, bf2):
    ei = edge_index.astype(jnp.int32)
    src4 = ei[0].reshape(NW, NH, HC, CH)
    dst3 = ei[1].reshape(NW, NCH, CH)
    dst4 = ei[1].reshape(NW, NH, HC, CH)
    tgt = target_index.astype(jnp.int32)
    xp = jnp.pad(x, ((0, NP - N), (0, 0)))
    b0r = b0.reshape(1, D)
    b1r = b1.reshape(1, D)
    gammar = gamma0.reshape(1, D)
    betar = beta0.reshape(1, D)
    bf1r = bf1.reshape(1, D)
    bf2r = bf2.reshape(1, D)

    zeros = jnp.zeros((NP, D), jnp.float32)

    ones_rows = jnp.ones((CH, D), jnp.float32)
    degp = _deg_kernel(dst3, ones_rows, zeros)
    dinvb = _dinv_call(degp)
    g0 = _mm_scale_call(xp, W0, dinvb)
    p0 = _agg_kernel(g0, src4, dst4, zeros)
    out0, s_sum, s_sq = _stats_call(p0, g0, dinvb, b0r)
    g1 = _bn_mm_call(out0, s_sum, s_sq, gammar, betar, W1, dinvb)
    p1 = _agg_kernel(g1, src4, dst4, zeros)
    out1 = _out1_call(p1, g1, dinvb, b1r)
    pooled = _tgt_gather_kernel(out1, tgt)
    return _ffn_call(pooled, Wf1, bf1r, Wf2, bf2r)


# speedup vs baseline: 26.0211x; 1.0229x over previous
import functools

import jax
import jax.numpy as jnp
from jax import lax
from jax.experimental import pallas as pl
from jax.experimental.pallas import tpu as pltpu
from jax.experimental.pallas import tpu_sc as plsc

N = 10000
NP = 10240
E = 320000
D = 128
B = 4096

NC = 2
NS = 16
NW = NC * NS
EPT = E // NW
CH = 125
NCH = EPT // CH
NH = 2
HC = NCH // NH
DW = 4
RPT = NP // NS
ZB = 128
BPT = B // NW

_MESH = plsc.VectorSubcoreMesh(core_axis_name="c", subcore_axis_name="s",
                               num_cores=NC, num_subcores=NS)


def _deg_body(dst_hbm, ones_hbm, zeros_hbm, out_hbm, degacc, dstv, ones_buf,
              dsem):
    cid = lax.axis_index("c")
    sid = lax.axis_index("s")
    wid = sid * NC + cid
    pltpu.sync_copy(dst_hbm.at[wid], dstv)
    pltpu.sync_copy(ones_hbm, ones_buf)
    pltpu.sync_copy(zeros_hbm.at[pl.ds(sid * RPT, RPT)],
                    degacc.at[pl.ds(sid * RPT, RPT)])
    plsc.subcore_barrier()

    def add_body(j, _):
        pltpu.async_copy(ones_buf, degacc.at[dstv.at[j]], dsem, add=True)

        @pl.when(j >= DW)
        def _():
            pltpu.make_async_copy(ones_buf, degacc.at[dstv.at[0]], dsem).wait()

        return 0

    lax.fori_loop(0, NCH, add_body, 0)
    for _ in range(DW):
        pltpu.make_async_copy(ones_buf, degacc.at[dstv.at[0]], dsem).wait()
    plsc.subcore_barrier()
    pltpu.sync_copy(degacc.at[pl.ds(sid * RPT, RPT)],
                    out_hbm.at[cid].at[pl.ds(sid * RPT, RPT)])


def _agg_body(g_hbm, src_hbm, dst_hbm, zeros_hbm, out_hbm,
              acc, srcv, dstv, rows0, rows1, sem0, sem1):
    cid = lax.axis_index("c")
    sid = lax.axis_index("s")
    wid = sid * NC + cid
    pltpu.sync_copy(zeros_hbm.at[pl.ds(sid * RPT, RPT)],
                    acc.at[pl.ds(sid * RPT, RPT)])
    plsc.subcore_barrier()

    for h in range(NH):
        pltpu.sync_copy(src_hbm.at[wid].at[h], srcv)
        pltpu.sync_copy(dst_hbm.at[wid].at[h], dstv)
        pltpu.async_copy(g_hbm.at[srcv.at[0]], rows0, sem0)

        def pair_body(k, _):
            pltpu.async_copy(g_hbm.at[srcv.at[2 * k + 1]], rows1, sem1)
            pltpu.make_async_copy(g_hbm.at[srcv.at[2 * k]], rows0, sem0).wait()
            pltpu.sync_copy(rows0, acc.at[dstv.at[2 * k]], add=True)

            @pl.when(k < HC // 2 - 1)
            def _():
                pltpu.async_copy(g_hbm.at[srcv.at[2 * k + 2]], rows0, sem0)

            pltpu.make_async_copy(g_hbm.at[srcv.at[2 * k + 1]], rows1,
                                  sem1).wait()
            pltpu.sync_copy(rows1, acc.at[dstv.at[2 * k + 1]], add=True)
            return 0

        lax.fori_loop(0, HC // 2, pair_body, 0)
    plsc.subcore_barrier()
    pltpu.sync_copy(acc.at[pl.ds(sid * RPT, RPT)],
                    out_hbm.at[cid].at[pl.ds(sid * RPT, RPT)])


def _tgt_gather_body(tab_hbm, idx_hbm, out_hbm, idxv, rows, sem):
    cid = lax.axis_index("c")
    sid = lax.axis_index("s")
    wid = sid * NC + cid
    base = wid * BPT
    pltpu.sync_copy(idx_hbm.at[pl.ds(base, BPT)], idxv)
    pltpu.async_copy(tab_hbm.at[idxv], rows, sem).wait()
    pltpu.sync_copy(rows, out_hbm.at[pl.ds(base, BPT)])


_DEG_SCRATCH = [
    pltpu.VMEM_SHARED((NP, D), jnp.float32),
    pltpu.VMEM((NCH, CH), jnp.int32),
    pltpu.VMEM((CH, D), jnp.float32),
    pltpu.SemaphoreType.DMA,
]
_AGG_SCRATCH = [
    pltpu.VMEM_SHARED((NP, D), jnp.float32),
    pltpu.VMEM((HC, CH), jnp.int32),
    pltpu.VMEM((HC, CH), jnp.int32),
    pltpu.VMEM((CH, D), jnp.float32),
    pltpu.VMEM((CH, D), jnp.float32),
    pltpu.SemaphoreType.DMA,
    pltpu.SemaphoreType.DMA,
]
_TGT_SCRATCH = [
    pltpu.VMEM((BPT,), jnp.int32),
    pltpu.VMEM((BPT, D), jnp.float32),
    pltpu.SemaphoreType.DMA,
]

_deg_kernel = pl.kernel(
    _deg_body, out_type=jax.ShapeDtypeStruct((NC, NP, D), jnp.float32),
    mesh=_MESH, scratch_types=_DEG_SCRATCH)
_agg_kernel = pl.kernel(
    _agg_body, out_type=jax.ShapeDtypeStruct((NC, NP, D), jnp.float32),
    mesh=_MESH, scratch_types=_AGG_SCRATCH)
_tgt_gather_kernel = pl.kernel(
    _tgt_gather_body, out_type=jax.ShapeDtypeStruct((B, D), jnp.float32),
    mesh=_MESH, scratch_types=_TGT_SCRATCH)


RB = 1024
GN = NP // RB
FB = 1024
GF = B // FB


def _mm_scale_body(x_ref, w_ref, degp_ref, g0_ref, dinvb_ref):
    dinv = lax.rsqrt(degp_ref[0] + degp_ref[1] + 1.0)
    dinvb_ref[...] = dinv
    h = jnp.dot(x_ref[...], w_ref[...], preferred_element_type=jnp.float32)
    g0_ref[...] = h * dinv


_mm_scale_call = pl.pallas_call(
    _mm_scale_body,
    grid=(GN,),
    in_specs=[
        pl.BlockSpec((RB, D), lambda i: (i, 0)),
        pl.BlockSpec((D, D), lambda i: (0, 0)),
        pl.BlockSpec((NC, RB, D), lambda i: (0, i, 0)),
    ],
    out_specs=[
        pl.BlockSpec((RB, D), lambda i: (i, 0)),
        pl.BlockSpec((RB, D), lambda i: (i, 0)),
    ],
    out_shape=[
        jax.ShapeDtypeStruct((NP, D), jnp.float32),
        jax.ShapeDtypeStruct((NP, D), jnp.float32),
    ],
)


def _stats_body(p_ref, g_ref, dinvb_ref, b_ref, out_ref, sum_ref, sq_ref):
    @pl.when(pl.program_id(0) == 0)
    def _():
        sum_ref[...] = jnp.zeros_like(sum_ref)
        sq_ref[...] = jnp.zeros_like(sq_ref)

    val = (p_ref[0] + p_ref[1] + g_ref[...]) * dinvb_ref[...] + b_ref[...]
    out_ref[...] = val
    row = (pl.program_id(0) * RB
           + lax.broadcasted_iota(jnp.int32, (RB, D), 0))
    vm = jnp.where(row < N, val, 0.0)
    sum_ref[...] += jnp.sum(vm, axis=0, keepdims=True)
    sq_ref[...] += jnp.sum(vm * vm, axis=0, keepdims=True)


_stats_call = pl.pallas_call(
    _stats_body,
    grid=(GN,),
    in_specs=[
        pl.BlockSpec((NC, RB, D), lambda i: (0, i, 0)),
        pl.BlockSpec((RB, D), lambda i: (i, 0)),
        pl.BlockSpec((RB, D), lambda i: (i, 0)),
        pl.BlockSpec((1, D), lambda i: (0, 0)),
    ],
    out_specs=[
        pl.BlockSpec((RB, D), lambda i: (i, 0)),
        pl.BlockSpec((1, D), lambda i: (0, 0)),
        pl.BlockSpec((1, D), lambda i: (0, 0)),
    ],
    out_shape=[
        jax.ShapeDtypeStruct((NP, D), jnp.float32),
        jax.ShapeDtypeStruct((1, D), jnp.float32),
        jax.ShapeDtypeStruct((1, D), jnp.float32),
    ],
)


def _bn_mm_body(x_ref, sum_ref, sq_ref, gamma_ref, beta_ref, w_ref,
                dinvb_ref, out_ref):
    mean = sum_ref[...] / N
    var = sq_ref[...] / N - mean * mean
    scale = gamma_ref[...] * lax.rsqrt(var + 1e-5)
    shift = beta_ref[...] - mean * scale
    h = jnp.maximum(x_ref[...] * scale + shift, 0.0)
    out_ref[...] = jnp.dot(h, w_ref[...],
                           preferred_element_type=jnp.float32) * dinvb_ref[...]


_bn_mm_call = pl.pallas_call(
    _bn_mm_body,
    grid=(GN,),
    in_specs=[
        pl.BlockSpec((RB, D), lambda i: (i, 0)),
        pl.BlockSpec((1, D), lambda i: (0, 0)),
        pl.BlockSpec((1, D), lambda i: (0, 0)),
        pl.BlockSpec((1, D), lambda i: (0, 0)),
        pl.BlockSpec((1, D), lambda i: (0, 0)),
        pl.BlockSpec((D, D), lambda i: (0, 0)),
        pl.BlockSpec((RB, D), lambda i: (i, 0)),
    ],
    out_specs=pl.BlockSpec((RB, D), lambda i: (i, 0)),
    out_shape=jax.ShapeDtypeStruct((NP, D), jnp.float32),
)


def _out1_body(p_ref, g_ref, dinvb_ref, b_ref, out_ref):
    out_ref[...] = ((p_ref[0] + p_ref[1] + g_ref[...]) * dinvb_ref[...]
                    + b_ref[...])


_out1_call = pl.pallas_call(
    _out1_body,
    grid=(GN,),
    in_specs=[
        pl.BlockSpec((NC, RB, D), lambda i: (0, i, 0)),
        pl.BlockSpec((RB, D), lambda i: (i, 0)),
        pl.BlockSpec((RB, D), lambda i: (i, 0)),
        pl.BlockSpec((1, D), lambda i: (0, 0)),
    ],
    out_specs=pl.BlockSpec((RB, D), lambda i: (i, 0)),
    out_shape=jax.ShapeDtypeStruct((NP, D), jnp.float32),
)


def _ffn_body(x_ref, w1_ref, b1_ref, w2_ref, b2_ref, out_ref):
    f = jnp.maximum(
        jnp.dot(x_ref[...], w1_ref[...], preferred_element_type=jnp.float32)
        + b1_ref[...], 0.0)
    out_ref[...] = (
        jnp.dot(f, w2_ref[...], preferred_element_type=jnp.float32)
        + b2_ref[...])


_ffn_call = pl.pallas_call(
    _ffn_body,
    grid=(GF,),
    in_specs=[
        pl.BlockSpec((FB, D), lambda i: (i, 0)),
        pl.BlockSpec((D, D), lambda i: (0, 0)),
        pl.BlockSpec((1, D), lambda i: (0, 0)),
        pl.BlockSpec((D, D), lambda i: (0, 0)),
        pl.BlockSpec((1, D), lambda i: (0, 0)),
    ],
    out_specs=pl.BlockSpec((FB, D), lambda i: (i, 0)),
    out_shape=jax.ShapeDtypeStruct((B, D), jnp.float32),
)


def kernel(x, edge_index, target_index, W0, b0, gamma0, beta0, W1, b1,
           Wf1, bf1, Wf2, bf2):
    ei = edge_index.astype(jnp.int32)
    src4 = ei[0].reshape(NW, NH, HC, CH)
    dst3 = ei[1].reshape(NW, NCH, CH)
    dst4 = ei[1].reshape(NW, NH, HC, CH)
    tgt = target_index.astype(jnp.int32)
    xp = jnp.pad(x, ((0, NP - N), (0, 0)))
    b0r = b0.reshape(1, D)
    b1r = b1.reshape(1, D)
    gammar = gamma0.reshape(1, D)
    betar = beta0.reshape(1, D)
    bf1r = bf1.reshape(1, D)
    bf2r = bf2.reshape(1, D)

    zeros = jnp.zeros((NP, D), jnp.float32)

    ones_rows = jnp.ones((CH, D), jnp.float32)
    degp = _deg_kernel(dst3, ones_rows, zeros)
    g0, dinvb = _mm_scale_call(xp, W0, degp)
    p0 = _agg_kernel(g0, src4, dst4, zeros)
    out0, s_sum, s_sq = _stats_call(p0, g0, dinvb, b0r)
    g1 = _bn_mm_call(out0, s_sum, s_sq, gammar, betar, W1, dinvb)
    p1 = _agg_kernel(g1, src4, dst4, zeros)
    out1 = _out1_call(p1, g1, dinvb, b1r)
    pooled = _tgt_gather_kernel(out1, tgt)
    return _ffn_call(pooled, Wf1, bf1r, Wf2, bf2r)
